# baseline (speedup 1.0000x reference)
import functools

import jax
import jax.numpy as jnp
import numpy as np
from jax.experimental import pallas as pl
from jax.experimental.pallas import tpu as pltpu

_H = 12
_E = 3
_TOP_P = 0.5
_ALPHA = 0.5
_EPS = 1e-10


def _i32(x):
    return x.astype(jnp.int32)


def _count_lt16(v16, cand16, L):
    ind = jnp.where(v16 < cand16, jnp.int16(1), jnp.int16(0))
    chunks = [ind[:, c * 128:(c + 1) * 128] for c in range(L // 128)]
    while len(chunks) > 1:
        chunks = [chunks[i] + chunks[i + 1] for i in range(0, len(chunks), 2)]
    return jnp.sum(chunks[0].astype(jnp.int32), axis=1, keepdims=True)


def _gelu_exact(a):
    return 0.5 * a * (1.0 + jax.lax.erf(a * np.float32(1.0 / np.sqrt(2.0))))


_IMIN = np.int32(-2**31)


def _fused(x2_ref, xh_ref, masks_ref, w1_ref, b1_ref, w2_ref, b2_ref, gw_ref,
           gcw_ref, gcb_ref, out_ref, loss_ref, xp_s, s0_s, acc_s,
           adj_a, adj_b, hi_a, hi_b,
           *, RB, NB, H, dh, L, k0):
    ib = pl.program_id(0)
    h = pl.program_id(1)
    lin = ib * H + h

    def produce(lin_t, adj_s, hi_s):
        lin_c = jnp.minimum(lin_t, NB * H - 1)
        ib2 = lin_c // H
        h2 = lin_c - ib2 * H
        x_rows = xh_ref[h2, pl.ds(ib2 * RB, RB), :]
        x_full = xh_ref[h2]
        p1 = jax.lax.dot_general(
            x_rows, w1_ref[...], (((1,), (1,)), ((), ())),
            preferred_element_type=jnp.float32) + b1_ref[...]
        p2 = jax.lax.dot_general(
            x_full, w2_ref[...], (((1,), (1,)), ((), ())),
            preferred_element_type=jnp.float32) + b2_ref[...]
        adj = _gelu_exact(jax.lax.dot_general(
            p1, p2, (((1,), (1,)), ((), ())),
            preferred_element_type=jnp.float32))
        bits = jax.lax.bitcast_convert_type(adj, jnp.int32)
        key = jnp.where(bits < 0, _IMIN + (jnp.int32(-1) - bits), bits)
        adj_s[...] = adj
        hi_s[...] = (jax.lax.shift_right_logical(key ^ _IMIN, 16)
                     - 32768).astype(jnp.int16)

    def consume(adj_s, hi_s):
        adj = adj_s[...]
        hi16 = hi_s[...]
        bits = jax.lax.bitcast_convert_type(adj, jnp.int32)
        key = jnp.where(bits < 0, _IMIN + (jnp.int32(-1) - bits), bits)
        prefix = jnp.zeros((RB, 1), jnp.int32)
        c_p = jnp.zeros((RB, 1), jnp.int32)
        for bpos in range(15, -1, -1):
            cand = prefix | (1 << bpos)
            cand16 = (cand - 32768).astype(jnp.int16)
            cnt = _count_lt16(hi16, cand16, L)
            take = cnt <= k0
            prefix = jnp.where(take, cand, prefix)
            c_p = jnp.where(take, cnt, c_p)
        r = k0 - c_p
        tie = hi16 == (prefix - 32768).astype(jnp.int16)
        lo_u = (key ^ _IMIN) & jnp.int32(0xffff)
        lo16 = jnp.where(tie, (lo_u - 32768).astype(jnp.int16),
                         jnp.int16(32767))
        prefix2 = jnp.zeros((RB, 1), jnp.int32)
        for bpos in range(15, -1, -1):
            cand = prefix2 | (1 << bpos)
            cand16 = (cand - 32768).astype(jnp.int16)
            cnt = _count_lt16(lo16, cand16, L)
            prefix2 = jnp.where(cnt <= r, cand, prefix2)
        thr_s = ((prefix << 16) | prefix2) ^ _IMIN
        adjm = jnp.where(key > thr_s, adj, 0.0)

        logits = jax.lax.dot_general(
            adjm, gw_ref[...], (((1,), (1,)), ((), ())),
            preferred_element_type=jnp.float32)
        mx = jnp.max(logits, axis=1, keepdims=True)
        ex = jnp.exp(logits - mx)
        probs = ex / jnp.sum(ex, axis=1, keepdims=True)
        ent = -jnp.sum(probs * jnp.log(probs + _EPS))

        p0 = probs[:, 0:1]
        p1g = probs[:, 1:2]
        p2g = probs[:, 2:3]
        r0 = _i32(p1g > p0) + _i32(p2g > p0)
        r1 = _i32(p0 >= p1g) + _i32(p2g > p1g)
        r2 = _i32(p0 >= p2g) + _i32(p1g >= p2g)
        sp0 = (jnp.where(r0 == 0, p0, 0.) + jnp.where(r1 == 0, p1g, 0.)
               + jnp.where(r2 == 0, p2g, 0.))
        sp1 = (jnp.where(r0 == 1, p0, 0.) + jnp.where(r1 == 1, p1g, 0.)
               + jnp.where(r2 == 1, p2g, 0.))
        sp2 = (jnp.where(r0 == 2, p0, 0.) + jnp.where(r1 == 2, p1g, 0.)
               + jnp.where(r2 == 2, p2g, 0.))
        keep1 = (sp0 <= _TOP_P).astype(jnp.float32)
        keep2 = ((sp0 + sp1) <= _TOP_P).astype(jnp.float32)

        def gate_of(rr):
            return (jnp.where(rr == 0, 1.0, 0.0)
                    + jnp.where(rr == 1, keep1, 0.0)
                    + jnp.where(rr == 2, keep2, 0.0))
        g0, g1, g2 = gate_of(r0), gate_of(r1), gate_of(r2)

        s0_s[...] = s0_s[...] + jnp.concatenate(
            [sp0, sp1 * keep1, sp2 * keep2], axis=1)
        acc_s[0] = acc_s[0] + ent

        mm = (g0 * masks_ref[:, 0, :] + g1 * masks_ref[:, 1, :]
              + g2 * masks_ref[:, 2, :])
        rowid = ib * RB + jax.lax.broadcasted_iota(jnp.int32, (RB, L), 0)
        colid = jax.lax.broadcasted_iota(jnp.int32, (RB, L), 1)
        mm = mm + jnp.where(rowid == colid, 1.0, 0.0)
        a2 = adjm * mm
        rmax = jnp.max(a2, axis=1, keepdims=True)
        e2 = jnp.exp(a2 - rmax)
        psm = e2 / jnp.sum(e2, axis=1, keepdims=True)

        out_ref[0] = jax.lax.dot_general(
            psm, xp_s[h], (((1,), (0,)), ((), ())),
            preferred_element_type=jnp.float32)

    @pl.when(lin == 0)
    def _init():
        x2 = x2_ref[...]
        for hh in range(H):
            w_h = gcw_ref[hh * dh:(hh + 1) * dh, :]
            xp_s[hh] = jax.lax.dot_general(
                x2, w_h, (((1,), (1,)), ((), ())),
                preferred_element_type=jnp.float32) + gcb_ref[hh]
        acc_s[0] = 0.0
        acc_s[1] = 0.0
        acc_s[2] = 0.0
        produce(jnp.int32(0), adj_a, hi_a)

    @pl.when(h == 0)
    def _init_s0():
        s0_s[...] = jnp.zeros_like(s0_s)

    par = lin - (lin // 2) * 2

    @pl.when(par == 0)
    def _even():
        produce(lin + 1, adj_b, hi_b)
        consume(adj_a, hi_a)

    @pl.when(par == 1)
    def _odd():
        produce(lin + 1, adj_a, hi_a)
        consume(adj_b, hi_b)

    @pl.when(h == H - 1)
    def _fin_s0():
        blk = s0_s[...]
        acc_s[1] = acc_s[1] + jnp.sum(blk)
        acc_s[2] = acc_s[2] + jnp.sum(blk * blk)

    @pl.when((ib == NB - 1) & (h == H - 1))
    def _fin():
        n = jnp.float32(L * 3)
        ssum = acc_s[1]
        mean = ssum / n
        var = (acc_s[2] - ssum * ssum / n) / (n - 1.0)
        loss_imp = var / (mean * mean + _EPS)
        loss_dyn = acc_s[0] / jnp.float32(H * 3)
        loss_ref[...] = jnp.full((1, 1), loss_imp + 0.1 * loss_dyn,
                                 jnp.float32)


def kernel(x, masks, proj1_W, proj1_b, proj2_W, proj2_b, gate_W, gcn_W, gcn_b):
    b, L, d = x.shape
    H = _H
    dh = d // H
    RB = 256
    NB = L // RB
    k0 = int(_ALPHA * L) - 1

    x2 = x.reshape(L, d)
    xh = x2.reshape(L, H, dh).transpose(1, 0, 2)
    b1 = proj1_b.reshape(1, dh)
    b2 = proj2_b.reshape(1, dh)
    gcb = gcn_b.reshape(H, 1, dh)

    grid = (NB, H)
    out3, loss = pl.pallas_call(
        functools.partial(_fused, RB=RB, NB=NB, H=H, dh=dh, L=L, k0=k0),
        grid=grid,
        in_specs=[
            pl.BlockSpec((L, d), lambda ib, h: (0, 0)),
            pl.BlockSpec((H, L, dh), lambda ib, h: (0, 0, 0)),
            pl.BlockSpec((RB, _E, L), lambda ib, h: (ib, 0, 0)),
            pl.BlockSpec((dh, dh), lambda ib, h: (0, 0)),
            pl.BlockSpec((1, dh), lambda ib, h: (0, 0)),
            pl.BlockSpec((dh, dh), lambda ib, h: (0, 0)),
            pl.BlockSpec((1, dh), lambda ib, h: (0, 0)),
            pl.BlockSpec((_E, L), lambda ib, h: (0, 0)),
            pl.BlockSpec((d, d), lambda ib, h: (0, 0)),
            pl.BlockSpec((H, 1, dh), lambda ib, h: (0, 0, 0)),
        ],
        out_specs=[
            pl.BlockSpec((1, RB, dh), lambda ib, h: (h, ib, 0)),
            pl.BlockSpec((1, 1), lambda ib, h: (0, 0)),
        ],
        out_shape=[
            jax.ShapeDtypeStruct((H, L, dh), jnp.float32),
            jax.ShapeDtypeStruct((1, 1), jnp.float32),
        ],
        scratch_shapes=[
            pltpu.VMEM((H, L, dh), jnp.float32),
            pltpu.VMEM((RB, _E), jnp.float32),
            pltpu.SMEM((4,), jnp.float32),
            pltpu.VMEM((RB, L), jnp.float32),
            pltpu.VMEM((RB, L), jnp.float32),
            pltpu.VMEM((RB, L), jnp.int16),
            pltpu.VMEM((RB, L), jnp.int16),
        ],
    )(x2, xh, masks, proj1_W, b1, proj2_W, b2, gate_W, gcn_W, gcb)

    out = out3.transpose(1, 0, 2).reshape(b, L, d)
    return out, loss.reshape(())


# speedup vs baseline: 2.4852x; 2.4852x over previous
import functools

import jax
import jax.numpy as jnp
import numpy as np
from jax.experimental import pallas as pl
from jax.experimental.pallas import tpu as pltpu

_H = 12
_E = 3
_TOP_P = 0.5
_ALPHA = 0.5
_EPS = 1e-10
_IMIN = np.int32(-2**31)


def _i32(x):
    return x.astype(jnp.int32)


def _count_lt16(v16, cand16, L):
    ind = jnp.where(v16 < cand16, jnp.int16(1), jnp.int16(0))
    chunks = [ind[:, c * 128:(c + 1) * 128] for c in range(L // 128)]
    while len(chunks) > 1:
        chunks = [chunks[i] + chunks[i + 1] for i in range(0, len(chunks), 2)]
    return jnp.sum(chunks[0].astype(jnp.int32), axis=1, keepdims=True)


def _gelu_exact(a):
    return 0.5 * a * (1.0 + jax.lax.erf(a * np.float32(1.0 / np.sqrt(2.0))))


def _keymap(bits):
    return jnp.where(bits < 0, _IMIN + (jnp.int32(-1) - bits), bits)


def _fused(x2_ref, xhf_ref, xhr_ref, masks_ref, w1_ref, b1_ref, w2_ref,
           b2_ref, gw_ref, gcw_ref, gcb_ref, out_ref, loss_ref,
           xp_s, s0_s, acc_s, adj_s, hi_s,
           *, RB, NB, H, dh, L, k0):
    ib = pl.program_id(0)
    h = pl.program_id(1)
    lin = ib * H + h

    def produce(x_rows, x_full):
        p1 = jax.lax.dot_general(
            x_rows, w1_ref[...], (((1,), (1,)), ((), ())),
            preferred_element_type=jnp.float32) + b1_ref[...]
        p2 = jax.lax.dot_general(
            x_full, w2_ref[...], (((1,), (1,)), ((), ())),
            preferred_element_type=jnp.float32) + b2_ref[...]
        adj = _gelu_exact(jax.lax.dot_general(
            p1, p2, (((1,), (1,)), ((), ())),
            preferred_element_type=jnp.float32))
        key = _keymap(jax.lax.bitcast_convert_type(adj, jnp.int32))
        adj_s[...] = adj
        hi_s[...] = (jax.lax.shift_right_logical(key ^ _IMIN, 16)
                     - 32768).astype(jnp.int16)

    @pl.when(lin == 0)
    def _init():
        x2 = x2_ref[...]
        for hh in range(H):
            w_h = gcw_ref[hh * dh:(hh + 1) * dh, :]
            xp_s[hh] = jax.lax.dot_general(
                x2, w_h, (((1,), (1,)), ((), ())),
                preferred_element_type=jnp.float32) + gcb_ref[hh]
        acc_s[0] = 0.0
        acc_s[1] = 0.0
        acc_s[2] = 0.0
        produce(x2_ref[0:RB, 0:dh], x2_ref[:, 0:dh])

    @pl.when(h == 0)
    def _init_s0():
        s0_s[...] = jnp.zeros_like(s0_s)

    adj = adj_s[...]
    hi16 = hi_s[...]
    key = _keymap(jax.lax.bitcast_convert_type(adj, jnp.int32))
    prefix = jnp.zeros((RB, 1), jnp.int32)
    c_p = jnp.zeros((RB, 1), jnp.int32)
    for bpos in range(15, -1, -1):
        cand = prefix | (1 << bpos)
        cand16 = (cand - 32768).astype(jnp.int16)
        cnt = _count_lt16(hi16, cand16, L)
        take = cnt <= k0
        prefix = jnp.where(take, cand, prefix)
        c_p = jnp.where(take, cnt, c_p)
    r = k0 - c_p
    tie = hi16 == (prefix - 32768).astype(jnp.int16)
    lo_u = (key ^ _IMIN) & jnp.int32(0xffff)
    lo16 = jnp.where(tie, (lo_u - 32768).astype(jnp.int16), jnp.int16(32767))
    prefix2 = jnp.zeros((RB, 1), jnp.int32)
    for bpos in range(15, -1, -1):
        cand = prefix2 | (1 << bpos)
        cand16 = (cand - 32768).astype(jnp.int16)
        cnt = _count_lt16(lo16, cand16, L)
        prefix2 = jnp.where(cnt <= r, cand, prefix2)
    thr_s = ((prefix << 16) | prefix2) ^ _IMIN
    adjm = jnp.where(key > thr_s, adj, 0.0)

    logits = jax.lax.dot_general(adjm, gw_ref[...], (((1,), (1,)), ((), ())),
                                 preferred_element_type=jnp.float32)
    mx = jnp.max(logits, axis=1, keepdims=True)
    ex = jnp.exp(logits - mx)
    probs = ex / jnp.sum(ex, axis=1, keepdims=True)
    ent = -jnp.sum(probs * jnp.log(probs + _EPS))

    p0 = probs[:, 0:1]
    p1g = probs[:, 1:2]
    p2g = probs[:, 2:3]
    r0 = _i32(p1g > p0) + _i32(p2g > p0)
    r1 = _i32(p0 >= p1g) + _i32(p2g > p1g)
    r2 = _i32(p0 >= p2g) + _i32(p1g >= p2g)
    sp0 = (jnp.where(r0 == 0, p0, 0.) + jnp.where(r1 == 0, p1g, 0.)
           + jnp.where(r2 == 0, p2g, 0.))
    sp1 = (jnp.where(r0 == 1, p0, 0.) + jnp.where(r1 == 1, p1g, 0.)
           + jnp.where(r2 == 1, p2g, 0.))
    sp2 = (jnp.where(r0 == 2, p0, 0.) + jnp.where(r1 == 2, p1g, 0.)
           + jnp.where(r2 == 2, p2g, 0.))
    keep1 = (sp0 <= _TOP_P).astype(jnp.float32)
    keep2 = ((sp0 + sp1) <= _TOP_P).astype(jnp.float32)

    def gate_of(rr):
        return (jnp.where(rr == 0, 1.0, 0.0) + jnp.where(rr == 1, keep1, 0.0)
                + jnp.where(rr == 2, keep2, 0.0))
    g0, g1, g2 = gate_of(r0), gate_of(r1), gate_of(r2)

    s0_s[...] = s0_s[...] + jnp.concatenate(
        [sp0, sp1 * keep1, sp2 * keep2], axis=1)
    acc_s[0] = acc_s[0] + ent

    mm = (g0 * masks_ref[:, 0, :] + g1 * masks_ref[:, 1, :]
          + g2 * masks_ref[:, 2, :])
    rowid = ib * RB + jax.lax.broadcasted_iota(jnp.int32, (RB, L), 0)
    colid = jax.lax.broadcasted_iota(jnp.int32, (RB, L), 1)
    mm = mm + jnp.where(rowid == colid, 1.0, 0.0)
    a2 = adjm * mm
    rmax = jnp.max(a2, axis=1, keepdims=True)
    e2 = jnp.exp(a2 - rmax)
    psm = e2 / jnp.sum(e2, axis=1, keepdims=True)

    out_ref[0] = jax.lax.dot_general(psm, xp_s[h], (((1,), (0,)), ((), ())),
                                     preferred_element_type=jnp.float32)

    produce(xhr_ref[0, 0], xhf_ref[0])

    @pl.when(h == H - 1)
    def _fin_s0():
        blk = s0_s[...]
        acc_s[1] = acc_s[1] + jnp.sum(blk)
        acc_s[2] = acc_s[2] + jnp.sum(blk * blk)

    @pl.when((ib == NB - 1) & (h == H - 1))
    def _fin():
        n = jnp.float32(L * 3)
        ssum = acc_s[1]
        mean = ssum / n
        var = (acc_s[2] - ssum * ssum / n) / (n - 1.0)
        loss_imp = var / (mean * mean + _EPS)
        loss_dyn = acc_s[0] / jnp.float32(H * 3)
        loss_ref[...] = jnp.full((1, 1), loss_imp + 0.1 * loss_dyn,
                                 jnp.float32)


def kernel(x, masks, proj1_W, proj1_b, proj2_W, proj2_b, gate_W, gcn_W, gcn_b):
    b, L, d = x.shape
    H = _H
    dh = d // H
    RB = 256
    NB = L // RB
    k0 = int(_ALPHA * L) - 1

    x2 = x.reshape(L, d)
    xh = x2.reshape(L, H, dh).transpose(1, 0, 2)
    xhr = xh.reshape(H, NB, RB, dh)
    b1 = proj1_b.reshape(1, dh)
    b2 = proj2_b.reshape(1, dh)
    gcb = gcn_b.reshape(H, 1, dh)

    last = NB * H - 1

    def _nxt(ib, h):
        l1 = jnp.minimum(ib * H + h + 1, last)
        ib2 = l1 // H
        return ib2, l1 - ib2 * H

    def _xhf_idx(ib, h):
        _, h2 = _nxt(ib, h)
        return (h2, 0, 0)

    def _xhr_idx(ib, h):
        ib2, h2 = _nxt(ib, h)
        return (h2, ib2, 0, 0)

    grid = (NB, H)
    out3, loss = pl.pallas_call(
        functools.partial(_fused, RB=RB, NB=NB, H=H, dh=dh, L=L, k0=k0),
        grid=grid,
        in_specs=[
            pl.BlockSpec((L, d), lambda ib, h: (0, 0)),
            pl.BlockSpec((1, L, dh), _xhf_idx),
            pl.BlockSpec((1, 1, RB, dh), _xhr_idx),
            pl.BlockSpec((RB, _E, L), lambda ib, h: (ib, 0, 0)),
            pl.BlockSpec((dh, dh), lambda ib, h: (0, 0)),
            pl.BlockSpec((1, dh), lambda ib, h: (0, 0)),
            pl.BlockSpec((dh, dh), lambda ib, h: (0, 0)),
            pl.BlockSpec((1, dh), lambda ib, h: (0, 0)),
            pl.BlockSpec((_E, L), lambda ib, h: (0, 0)),
            pl.BlockSpec((d, d), lambda ib, h: (0, 0)),
            pl.BlockSpec((H, 1, dh), lambda ib, h: (0, 0, 0)),
        ],
        out_specs=[
            pl.BlockSpec((1, RB, dh), lambda ib, h: (h, ib, 0)),
            pl.BlockSpec((1, 1), lambda ib, h: (0, 0)),
        ],
        out_shape=[
            jax.ShapeDtypeStruct((H, L, dh), jnp.float32),
            jax.ShapeDtypeStruct((1, 1), jnp.float32),
        ],
        scratch_shapes=[
            pltpu.VMEM((H, L, dh), jnp.float32),
            pltpu.VMEM((RB, _E), jnp.float32),
            pltpu.SMEM((4,), jnp.float32),
            pltpu.VMEM((RB, L), jnp.float32),
            pltpu.VMEM((RB, L), jnp.int16),
        ],
    )(x2, xh, xhr, masks, proj1_W, b1, proj2_W, b2, gate_W, gcn_W, gcb)

    out = out3.transpose(1, 0, 2).reshape(b, L, d)
    return out, loss.reshape(())


# speedup vs baseline: 3.0812x; 1.2398x over previous
import functools

import jax
import jax.numpy as jnp
import numpy as np
from jax.experimental import pallas as pl
from jax.experimental.pallas import tpu as pltpu

_H = 12
_E = 3
_TOP_P = 0.5
_ALPHA = 0.5
_EPS = 1e-10


def _i32(x):
    return x.astype(jnp.int32)


def _sum_chunks16(ind, L):
    chunks = [ind[:, c * 128:(c + 1) * 128] for c in range(L // 128)]
    while len(chunks) > 1:
        chunks = [chunks[i] + chunks[i + 1] for i in range(0, len(chunks), 2)]
    return jnp.sum(chunks[0].astype(jnp.int32), axis=1, keepdims=True)


def _count_lt16(v16, cand16, L):
    return _sum_chunks16(jnp.where(v16 < cand16, jnp.int16(1), jnp.int16(0)),
                         L)


def _row_min16(v16, L):
    return jnp.min(v16.astype(jnp.int32), axis=1, keepdims=True)


def _gelu_exact(a):
    return 0.5 * a * (1.0 + jax.lax.erf(a * np.float32(1.0 / np.sqrt(2.0))))


def _fused(x2_ref, xh_ref, masks_ref, w1_ref, b1_ref, w2_ref, b2_ref, gw_ref,
           gcw_ref, gcb_ref, out_ref, loss_ref, xp_s, s0_s, acc_s, thrlo_s,
           *, RB, NB, H, dh, L, k0):
    ib = pl.program_id(0)
    h = pl.program_id(1)

    @pl.when((ib == 0) & (h == 0))
    def _init():
        x2 = x2_ref[...]
        for hh in range(H):
            w_h = gcw_ref[hh * dh:(hh + 1) * dh, :]
            xp_s[hh] = jax.lax.dot_general(
                x2, w_h, (((1,), (1,)), ((), ())),
                preferred_element_type=jnp.float32) + gcb_ref[hh]
        acc_s[0] = 0.0
        acc_s[1] = 0.0
        acc_s[2] = 0.0

    @pl.when(h == 0)
    def _init_s0():
        s0_s[...] = jnp.zeros_like(s0_s)

    x_h = xh_ref[0]
    x_rows = xh_ref[0, pl.ds(ib * RB, RB), :]
    p1 = jax.lax.dot_general(x_rows, w1_ref[...], (((1,), (1,)), ((), ())),
                             preferred_element_type=jnp.float32) + b1_ref[...]
    p2 = jax.lax.dot_general(x_h, w2_ref[...], (((1,), (1,)), ((), ())),
                             preferred_element_type=jnp.float32) + b2_ref[...]
    adj = _gelu_exact(jax.lax.dot_general(
        p1, p2, (((1,), (1,)), ((), ())), preferred_element_type=jnp.float32))

    bits = jax.lax.bitcast_convert_type(adj, jnp.int32)
    imin = jnp.int32(-2**31)
    key = jnp.where(bits < 0, imin + (jnp.int32(-1) - bits), bits)
    key_u = key ^ imin
    hi_u = jax.lax.shift_right_logical(key_u, 16)
    hi16 = (hi_u - 32768).astype(jnp.int16)
    prefix = jnp.zeros((RB, 1), jnp.int32)
    c_p = jnp.zeros((RB, 1), jnp.int32)
    for bpos in range(15, -1, -1):
        cand = prefix | (1 << bpos)
        cand16 = (cand - 32768).astype(jnp.int16)
        cnt = _count_lt16(hi16, cand16, L)
        take = cnt <= k0
        prefix = jnp.where(take, cand, prefix)
        c_p = jnp.where(take, cnt, c_p)
    r = k0 - c_p
    tie = hi16 == (prefix - 32768).astype(jnp.int16)
    lo_u = key_u & jnp.int32(0xffff)
    lo16 = jnp.where(tie, (lo_u - 32768).astype(jnp.int16), jnp.int16(32767))
    rem = r
    resolved = jnp.zeros((RB, 1), jnp.int32)
    ans = jnp.zeros((RB, 1), jnp.int32)
    cur = lo16
    for _ in range(4):
        m = _row_min16(cur, L)
        m16 = m.astype(jnp.int16)
        eqm = cur == m16
        c = _sum_chunks16(jnp.where(eqm, jnp.int16(1), jnp.int16(0)), L)
        newly = (resolved == 0) & (rem < c)
        ans = jnp.where(newly, m + 32768, ans)
        resolved = resolved | newly.astype(jnp.int32)
        rem = jnp.where(resolved > 0, rem, rem - c)
        cur = jnp.where(eqm, jnp.int16(32767), cur)
    nbad = jnp.sum(1 - resolved)
    thrlo_s[...] = ans

    @pl.when(nbad > 0)
    def _stage2_full():
        prefix2 = jnp.zeros((RB, 1), jnp.int32)
        for bpos in range(15, -1, -1):
            cand = prefix2 | (1 << bpos)
            cand16 = (cand - 32768).astype(jnp.int16)
            cnt = _count_lt16(lo16, cand16, L)
            prefix2 = jnp.where(cnt <= r, cand, prefix2)
        thrlo_s[...] = prefix2

    thr_s = ((prefix << 16) | thrlo_s[...]) ^ imin
    adjm = jnp.where(key > thr_s, adj, 0.0)

    logits = jax.lax.dot_general(adjm, gw_ref[...], (((1,), (1,)), ((), ())),
                                 preferred_element_type=jnp.float32)
    mx = jnp.max(logits, axis=1, keepdims=True)
    ex = jnp.exp(logits - mx)
    probs = ex / jnp.sum(ex, axis=1, keepdims=True)
    ent = -jnp.sum(probs * jnp.log(probs + _EPS))

    p0 = probs[:, 0:1]
    p1g = probs[:, 1:2]
    p2g = probs[:, 2:3]
    r0 = _i32(p1g > p0) + _i32(p2g > p0)
    r1 = _i32(p0 >= p1g) + _i32(p2g > p1g)
    r2 = _i32(p0 >= p2g) + _i32(p1g >= p2g)
    sp0 = jnp.where(r0 == 0, p0, 0.) + jnp.where(r1 == 0, p1g, 0.) + jnp.where(r2 == 0, p2g, 0.)
    sp1 = jnp.where(r0 == 1, p0, 0.) + jnp.where(r1 == 1, p1g, 0.) + jnp.where(r2 == 1, p2g, 0.)
    sp2 = jnp.where(r0 == 2, p0, 0.) + jnp.where(r1 == 2, p1g, 0.) + jnp.where(r2 == 2, p2g, 0.)
    keep1 = (sp0 <= _TOP_P).astype(jnp.float32)
    keep2 = ((sp0 + sp1) <= _TOP_P).astype(jnp.float32)

    def gate_of(r):
        return (jnp.where(r == 0, 1.0, 0.0) + jnp.where(r == 1, keep1, 0.0)
                + jnp.where(r == 2, keep2, 0.0))
    g0, g1, g2 = gate_of(r0), gate_of(r1), gate_of(r2)

    s0_s[...] = s0_s[...] + jnp.concatenate(
        [sp0, sp1 * keep1, sp2 * keep2], axis=1)
    acc_s[0] = acc_s[0] + ent

    mm = (g0 * masks_ref[:, 0, :] + g1 * masks_ref[:, 1, :]
          + g2 * masks_ref[:, 2, :])
    rowid = ib * RB + jax.lax.broadcasted_iota(jnp.int32, (RB, L), 0)
    colid = jax.lax.broadcasted_iota(jnp.int32, (RB, L), 1)
    mm = mm + jnp.where(rowid == colid, 1.0, 0.0)
    a2 = adjm * mm
    rmax = jnp.max(a2, axis=1, keepdims=True)
    e2 = jnp.exp(a2 - rmax)
    psm = e2 / jnp.sum(e2, axis=1, keepdims=True)

    out_ref[0] = jax.lax.dot_general(psm, xp_s[h], (((1,), (0,)), ((), ())),
                                     preferred_element_type=jnp.float32)

    @pl.when(h == H - 1)
    def _fin_s0():
        blk = s0_s[...]
        acc_s[1] = acc_s[1] + jnp.sum(blk)
        acc_s[2] = acc_s[2] + jnp.sum(blk * blk)

    @pl.when((ib == NB - 1) & (h == H - 1))
    def _fin():
        n = jnp.float32(L * 3)
        ssum = acc_s[1]
        mean = ssum / n
        var = (acc_s[2] - ssum * ssum / n) / (n - 1.0)
        loss_imp = var / (mean * mean + _EPS)
        loss_dyn = acc_s[0] / jnp.float32(H * 3)
        loss_ref[...] = jnp.full((1, 1), loss_imp + 0.1 * loss_dyn,
                                 jnp.float32)


def kernel(x, masks, proj1_W, proj1_b, proj2_W, proj2_b, gate_W, gcn_W, gcn_b):
    b, L, d = x.shape
    H = _H
    dh = d // H
    RB = 256
    NB = L // RB
    k0 = int(_ALPHA * L) - 1

    x2 = x.reshape(L, d)
    xh = x2.reshape(L, H, dh).transpose(1, 0, 2)
    b1 = proj1_b.reshape(1, dh)
    b2 = proj2_b.reshape(1, dh)
    gcb = gcn_b.reshape(H, 1, dh)

    grid = (NB, H)
    out3, loss = pl.pallas_call(
        functools.partial(_fused, RB=RB, NB=NB, H=H, dh=dh, L=L, k0=k0),
        grid=grid,
        in_specs=[
            pl.BlockSpec((L, d), lambda ib, h: (0, 0)),
            pl.BlockSpec((1, L, dh), lambda ib, h: (h, 0, 0)),
            pl.BlockSpec((RB, _E, L), lambda ib, h: (ib, 0, 0)),
            pl.BlockSpec((dh, dh), lambda ib, h: (0, 0)),
            pl.BlockSpec((1, dh), lambda ib, h: (0, 0)),
            pl.BlockSpec((dh, dh), lambda ib, h: (0, 0)),
            pl.BlockSpec((1, dh), lambda ib, h: (0, 0)),
            pl.BlockSpec((_E, L), lambda ib, h: (0, 0)),
            pl.BlockSpec((d, d), lambda ib, h: (0, 0)),
            pl.BlockSpec((H, 1, dh), lambda ib, h: (0, 0, 0)),
        ],
        out_specs=[
            pl.BlockSpec((1, RB, dh), lambda ib, h: (h, ib, 0)),
            pl.BlockSpec((1, 1), lambda ib, h: (0, 0)),
        ],
        out_shape=[
            jax.ShapeDtypeStruct((H, L, dh), jnp.float32),
            jax.ShapeDtypeStruct((1, 1), jnp.float32),
        ],
        scratch_shapes=[
            pltpu.VMEM((H, L, dh), jnp.float32),
            pltpu.VMEM((RB, _E), jnp.float32),
            pltpu.SMEM((4,), jnp.float32),
            pltpu.VMEM((RB, 1), jnp.int32),
        ],
    )(x2, xh, masks, proj1_W, b1, proj2_W, b2, gate_W, gcn_W, gcb)

    out = out3.transpose(1, 0, 2).reshape(b, L, d)
    return out, loss.reshape(())


# speedup vs baseline: 3.2381x; 1.0509x over previous
import functools

import jax
import jax.numpy as jnp
import numpy as np
from jax.experimental import pallas as pl
from jax.experimental.pallas import tpu as pltpu

_H = 12
_E = 3
_TOP_P = 0.5
_ALPHA = 0.5
_EPS = 1e-10


def _i32(x):
    return x.astype(jnp.int32)


def _sum_chunks16(ind, L):
    chunks = [ind[:, c * 128:(c + 1) * 128] for c in range(L // 128)]
    while len(chunks) > 1:
        chunks = [chunks[i] + chunks[i + 1] for i in range(0, len(chunks), 2)]
    return jnp.sum(chunks[0].astype(jnp.int32), axis=1, keepdims=True)


def _count_lt16(v16, cand16, L):
    return _sum_chunks16(jnp.where(v16 < cand16, jnp.int16(1), jnp.int16(0)),
                         L)


def _row_min16(v16, L):
    return jnp.min(v16.astype(jnp.int32), axis=1, keepdims=True)


def _gelu_exact(a):
    return 0.5 * a * (1.0 + jax.lax.erf(a * np.float32(1.0 / np.sqrt(2.0))))


def _fused(x2_ref, xh_ref, masks_ref, w1_ref, b1_ref, w2_ref, b2_ref, gw_ref,
           gcw_ref, gcb_ref, out_ref, loss_ref, xp_s, s0_s, acc_s, thrlo_s,
           diag_s, *, RB, NB, H, dh, L, k0):
    ib = pl.program_id(0)
    h = pl.program_id(1)

    @pl.when((ib == 0) & (h == 0))
    def _init():
        x2 = x2_ref[...]
        for hh in range(H):
            w_h = gcw_ref[hh * dh:(hh + 1) * dh, :]
            xp_s[hh] = jax.lax.dot_general(
                x2, w_h, (((1,), (1,)), ((), ())),
                preferred_element_type=jnp.float32) + gcb_ref[hh]
        acc_s[0] = 0.0
        acc_s[1] = 0.0
        acc_s[2] = 0.0

    @pl.when(h == 0)
    def _init_s0():
        s0_s[...] = jnp.zeros_like(s0_s)
        rowid = ib * RB + jax.lax.broadcasted_iota(jnp.int32, (RB, L), 0)
        colid = jax.lax.broadcasted_iota(jnp.int32, (RB, L), 1)
        diag_s[...] = jnp.where(rowid == colid, 1.0, 0.0)

    x_h = xh_ref[0]
    x_rows = xh_ref[0, pl.ds(ib * RB, RB), :]
    p1 = jax.lax.dot_general(x_rows, w1_ref[...], (((1,), (1,)), ((), ())),
                             preferred_element_type=jnp.float32) + b1_ref[...]
    p2 = jax.lax.dot_general(x_h, w2_ref[...], (((1,), (1,)), ((), ())),
                             preferred_element_type=jnp.float32) + b2_ref[...]
    adj = _gelu_exact(jax.lax.dot_general(
        p1, p2, (((1,), (1,)), ((), ())), preferred_element_type=jnp.float32))

    bits = jax.lax.bitcast_convert_type(adj, jnp.int32)
    imin = jnp.int32(-2**31)
    key = jnp.where(bits < 0, imin + (jnp.int32(-1) - bits), bits)
    key_u = key ^ imin
    hi_u = jax.lax.shift_right_logical(key_u, 16)
    hi16 = (hi_u - 32768).astype(jnp.int16)
    prefix = jnp.zeros((RB, 1), jnp.int32)
    c_p = jnp.zeros((RB, 1), jnp.int32)
    for bpos in range(15, -1, -1):
        cand = prefix | (1 << bpos)
        cand16 = (cand - 32768).astype(jnp.int16)
        cnt = _count_lt16(hi16, cand16, L)
        take = cnt <= k0
        prefix = jnp.where(take, cand, prefix)
        c_p = jnp.where(take, cnt, c_p)
    r = k0 - c_p
    tie = hi16 == (prefix - 32768).astype(jnp.int16)
    lo_u = key_u & jnp.int32(0xffff)
    lo16 = jnp.where(tie, (lo_u - 32768).astype(jnp.int16), jnp.int16(32767))
    rem = r
    resolved = jnp.zeros((RB, 1), jnp.int32)
    ans = jnp.zeros((RB, 1), jnp.int32)
    cur = lo16
    for _ in range(3):
        m = _row_min16(cur, L)
        m16 = m.astype(jnp.int16)
        eqm = cur == m16
        c = _sum_chunks16(jnp.where(eqm, jnp.int16(1), jnp.int16(0)), L)
        newly = (resolved == 0) & (rem < c)
        ans = jnp.where(newly, m + 32768, ans)
        resolved = resolved | newly.astype(jnp.int32)
        rem = jnp.where(resolved > 0, rem, rem - c)
        cur = jnp.where(eqm, jnp.int16(32767), cur)
    nbad = jnp.sum(1 - resolved)
    thrlo_s[...] = ans

    @pl.when(nbad > 0)
    def _stage2_full():
        prefix2 = jnp.zeros((RB, 1), jnp.int32)
        for bpos in range(15, -1, -1):
            cand = prefix2 | (1 << bpos)
            cand16 = (cand - 32768).astype(jnp.int16)
            cnt = _count_lt16(lo16, cand16, L)
            prefix2 = jnp.where(cnt <= r, cand, prefix2)
        thrlo_s[...] = prefix2

    thr_s = ((prefix << 16) | thrlo_s[...]) ^ imin
    adjm = jnp.where(key > thr_s, adj, 0.0)

    logits = jax.lax.dot_general(adjm, gw_ref[...], (((1,), (1,)), ((), ())),
                                 preferred_element_type=jnp.float32)
    mx = jnp.max(logits, axis=1, keepdims=True)
    ex = jnp.exp(logits - mx)
    probs = ex / jnp.sum(ex, axis=1, keepdims=True)
    ent = -jnp.sum(probs * jnp.log(probs + _EPS))

    p0 = probs[:, 0:1]
    p1g = probs[:, 1:2]
    p2g = probs[:, 2:3]
    r0 = _i32(p1g > p0) + _i32(p2g > p0)
    r1 = _i32(p0 >= p1g) + _i32(p2g > p1g)
    r2 = _i32(p0 >= p2g) + _i32(p1g >= p2g)
    sp0 = jnp.where(r0 == 0, p0, 0.) + jnp.where(r1 == 0, p1g, 0.) + jnp.where(r2 == 0, p2g, 0.)
    sp1 = jnp.where(r0 == 1, p0, 0.) + jnp.where(r1 == 1, p1g, 0.) + jnp.where(r2 == 1, p2g, 0.)
    sp2 = jnp.where(r0 == 2, p0, 0.) + jnp.where(r1 == 2, p1g, 0.) + jnp.where(r2 == 2, p2g, 0.)
    keep1 = (sp0 <= _TOP_P).astype(jnp.float32)
    keep2 = ((sp0 + sp1) <= _TOP_P).astype(jnp.float32)

    def gate_of(r):
        return (jnp.where(r == 0, 1.0, 0.0) + jnp.where(r == 1, keep1, 0.0)
                + jnp.where(r == 2, keep2, 0.0))
    g0, g1, g2 = gate_of(r0), gate_of(r1), gate_of(r2)

    s0_s[...] = s0_s[...] + jnp.concatenate(
        [sp0, sp1 * keep1, sp2 * keep2], axis=1)
    acc_s[0] = acc_s[0] + ent

    mm = (g0 * masks_ref[:, 0, :] + g1 * masks_ref[:, 1, :]
          + g2 * masks_ref[:, 2, :]) + diag_s[...]
    a2 = adjm * mm
    rmax = jnp.max(a2, axis=1, keepdims=True)
    e2 = jnp.exp(a2 - rmax)
    psm = e2 / jnp.sum(e2, axis=1, keepdims=True)

    out_ref[0] = jax.lax.dot_general(psm, xp_s[h], (((1,), (0,)), ((), ())),
                                     preferred_element_type=jnp.float32)

    @pl.when(h == H - 1)
    def _fin_s0():
        blk = s0_s[...]
        acc_s[1] = acc_s[1] + jnp.sum(blk)
        acc_s[2] = acc_s[2] + jnp.sum(blk * blk)

    @pl.when((ib == NB - 1) & (h == H - 1))
    def _fin():
        n = jnp.float32(L * 3)
        ssum = acc_s[1]
        mean = ssum / n
        var = (acc_s[2] - ssum * ssum / n) / (n - 1.0)
        loss_imp = var / (mean * mean + _EPS)
        loss_dyn = acc_s[0] / jnp.float32(H * 3)
        loss_ref[...] = jnp.full((1, 1), loss_imp + 0.1 * loss_dyn,
                                 jnp.float32)


def kernel(x, masks, proj1_W, proj1_b, proj2_W, proj2_b, gate_W, gcn_W, gcn_b):
    b, L, d = x.shape
    H = _H
    dh = d // H
    RB = 256
    NB = L // RB
    k0 = int(_ALPHA * L) - 1

    x2 = x.reshape(L, d)
    xh = x2.reshape(L, H, dh).transpose(1, 0, 2)
    b1 = proj1_b.reshape(1, dh)
    b2 = proj2_b.reshape(1, dh)
    gcb = gcn_b.reshape(H, 1, dh)

    grid = (NB, H)
    out3, loss = pl.pallas_call(
        functools.partial(_fused, RB=RB, NB=NB, H=H, dh=dh, L=L, k0=k0),
        grid=grid,
        in_specs=[
            pl.BlockSpec((L, d), lambda ib, h: (0, 0)),
            pl.BlockSpec((1, L, dh), lambda ib, h: (h, 0, 0)),
            pl.BlockSpec((RB, _E, L), lambda ib, h: (ib, 0, 0)),
            pl.BlockSpec((dh, dh), lambda ib, h: (0, 0)),
            pl.BlockSpec((1, dh), lambda ib, h: (0, 0)),
            pl.BlockSpec((dh, dh), lambda ib, h: (0, 0)),
            pl.BlockSpec((1, dh), lambda ib, h: (0, 0)),
            pl.BlockSpec((_E, L), lambda ib, h: (0, 0)),
            pl.BlockSpec((d, d), lambda ib, h: (0, 0)),
            pl.BlockSpec((H, 1, dh), lambda ib, h: (0, 0, 0)),
        ],
        out_specs=[
            pl.BlockSpec((1, RB, dh), lambda ib, h: (h, ib, 0)),
            pl.BlockSpec((1, 1), lambda ib, h: (0, 0)),
        ],
        out_shape=[
            jax.ShapeDtypeStruct((H, L, dh), jnp.float32),
            jax.ShapeDtypeStruct((1, 1), jnp.float32),
        ],
        scratch_shapes=[
            pltpu.VMEM((H, L, dh), jnp.float32),
            pltpu.VMEM((RB, _E), jnp.float32),
            pltpu.SMEM((4,), jnp.float32),
            pltpu.VMEM((RB, 1), jnp.int32),
            pltpu.VMEM((RB, L), jnp.float32),
        ],
    )(x2, xh, masks, proj1_W, b1, proj2_W, b2, gate_W, gcn_W, gcb)

    out = out3.transpose(1, 0, 2).reshape(b, L, d)
    return out, loss.reshape(())
